# Initial kernel scaffold; baseline (speedup 1.0000x reference)
import jax
import jax.numpy as jnp
from jax.experimental import pallas as pl


def kernel(idx, targets, table):
    raise NotImplementedError("write your pallas kernel here")


# speedup vs baseline: 1.3891x; 1.3891x over previous
import dataclasses
import functools

import jax
import jax.numpy as jnp
from jax import lax
from jax.experimental import pallas as pl
from jax.experimental.pallas import tpu as pltpu
from jax.experimental.pallas import tpu_sc as plsc

_L = 16
_NC, _NS = 2, 16
_NW = _NC * _NS
_W = 32


def _lse_body(tab_ref, lse_ref):
    x = tab_ref[...]
    m = jnp.max(x, axis=1, keepdims=True)
    s = jnp.sum(jnp.exp(x - m), axis=1, keepdims=True)
    lse_ref[...] = jnp.log(s) + m


def _loss_body(n, part_ref, loss_ref):
    loss_ref[...] = (jnp.sum(part_ref[...]) / jnp.float32(n)).reshape(1, 1)


@functools.lru_cache(maxsize=None)
def _make_sc_main(n, v):
    mesh = plsc.VectorSubcoreMesh(core_axis_name="c", subcore_axis_name="s")
    cp = pltpu.CompilerParams()
    for _f, _v in (("needs_layout_passes", False),
                   ("use_tc_tiling_on_sc", False)):
        if _f in pltpu.CompilerParams.__dataclass_fields__:
            cp = dataclasses.replace(cp, **{_f: _v})

    @functools.partial(
        pl.kernel,
        compiler_params=cp,
        out_type=[
            jax.ShapeDtypeStruct((n, v), jnp.float32),
            jax.ShapeDtypeStruct((_NW, _L), jnp.float32),
        ],
        mesh=mesh,
        scratch_types=[
            pltpu.VMEM((v,), jnp.float32),
            pltpu.VMEM((_L,), jnp.float32),
        ],
    )
    def sc_main(table_hbm, idx_hbm, tgt_hbm, lse_hbm, out_hbm, part_hbm,
                lse_v, acc_v):
        pltpu.sync_copy(lse_hbm, lse_v)
        acc_v[...] = jnp.zeros((_L,), jnp.float32)

        def body(i_vmem, t_vmem, o_vmem):
            pltpu.sync_copy(table_hbm.at[i_vmem.at[0]], o_vmem)

            @pl.loop(0, _W, step=_L)
            def _(j):
                rvec = j + lax.iota(jnp.int32, _L)
                ivec = i_vmem[0, pl.ds(j, _L)]
                tvec = t_vmem[0, pl.ds(j, _L)]
                lse_vals = plsc.load_gather(lse_v, [ivec])
                elems = plsc.load_gather(o_vmem, [rvec, tvec])
                acc_v[...] = acc_v[...] + (lse_vals - elems)

        pltpu.emit_pipeline(
            body,
            grid=(n // _W,),
            in_specs=[
                pl.BlockSpec((1, _W), lambda i: (0, i)),
                pl.BlockSpec((1, _W), lambda i: (0, i)),
            ],
            out_specs=[pl.BlockSpec((_W, v), lambda i: (i, 0))],
            core_axis_name=("c", "s"),
            dimension_semantics=(pltpu.PARALLEL,),
        )(idx_hbm, tgt_hbm, out_hbm)

        wid = lax.axis_index("s") * _NC + lax.axis_index("c")
        pltpu.sync_copy(acc_v, part_hbm.at[wid])

    return sc_main


def kernel(idx, targets, table):
    b, t = idx.shape
    v = table.shape[0]
    n = b * t
    idx_f = idx.reshape(1, n).astype(jnp.int32)
    tgt_f = targets.reshape(1, n).astype(jnp.int32)

    lse = pl.pallas_call(
        _lse_body,
        out_shape=jax.ShapeDtypeStruct((v, 1), jnp.float32),
    )(table)

    logits, partials = _make_sc_main(n, v)(table, idx_f, tgt_f, lse.reshape(v))

    loss = pl.pallas_call(
        functools.partial(_loss_body, n),
        out_shape=jax.ShapeDtypeStruct((1, 1), jnp.float32),
    )(partials)

    return loss[0, 0], logits


# speedup vs baseline: 2.3708x; 1.7068x over previous
import dataclasses
import functools

import jax
import jax.numpy as jnp
from jax import lax
from jax.experimental import pallas as pl
from jax.experimental.pallas import tpu as pltpu
from jax.experimental.pallas import tpu_sc as plsc

_L = 16
_NC, _NS = 2, 16
_NW = _NC * _NS
_W = 32
_RING = 2


def _lse_body(tab_ref, lse_ref):
    x = tab_ref[...]
    m = jnp.max(x, axis=1, keepdims=True)
    s = jnp.sum(jnp.exp(x - m), axis=1, keepdims=True)
    lse_ref[...] = jnp.log(s) + m


def _loss_body(n, part_ref, loss_ref):
    loss_ref[...] = (jnp.sum(part_ref[...]) / jnp.float32(n)).reshape(1, 1)


@functools.lru_cache(maxsize=None)
def _make_sc_main(n, v):
    mesh = plsc.VectorSubcoreMesh(core_axis_name="c", subcore_axis_name="s")
    cp = pltpu.CompilerParams()
    for _f, _v in (("needs_layout_passes", False),
                   ("use_tc_tiling_on_sc", True)):
        if _f in pltpu.CompilerParams.__dataclass_fields__:
            cp = dataclasses.replace(cp, **{_f: _v})

    ni = n // _NW
    steps = ni // _W
    assert steps % _RING == 0

    @functools.partial(
        pl.kernel,
        compiler_params=cp,
        out_type=[
            jax.ShapeDtypeStruct((n, 1024), jnp.float32),
            jax.ShapeDtypeStruct((_NW, 128), jnp.float32),
        ],
        mesh=mesh,
        scratch_types=[
            pltpu.VMEM((ni,), jnp.int32),
            pltpu.VMEM((ni,), jnp.int32),
            pltpu.VMEM((1024,), jnp.float32),
            pltpu.VMEM((128,), jnp.float32),
            pltpu.VMEM((_W, 1024), jnp.float32),
            pltpu.VMEM((_W, 1024), jnp.float32),
            pltpu.SemaphoreType.DMA,
            pltpu.SemaphoreType.DMA,
        ],
    )
    def sc_main(table_hbm, idx_hbm, tgt_hbm, lse_hbm, out_hbm, part_hbm,
                idx_v, tgt_v, lse_v, acc_v, buf0, buf1, sem0, sem1):
        wid = lax.axis_index("s") * _NC + lax.axis_index("c")
        base = wid * ni
        pltpu.sync_copy(idx_hbm.at[pl.ds(base, ni)], idx_v)
        pltpu.sync_copy(tgt_hbm.at[pl.ds(base, ni)], tgt_v)
        pltpu.sync_copy(lse_hbm, lse_v)

        @pl.loop(0, 128, step=_L)
        def _(j):
            acc_v[pl.ds(j, _L)] = jnp.zeros((_L,), jnp.float32)

        bufs = (buf0, buf1)
        sems = (sem0, sem1)

        @pl.loop(0, steps // _RING)
        def _(g):
            for b in range(_RING):
                step = g * _RING + b
                buf, sem = bufs[b], sems[b]

                @pl.when(g > 0)
                def _():
                    pltpu.make_async_copy(
                        buf,
                        out_hbm.at[pl.ds(base + (step - _RING) * _W, _W)],
                        sem,
                    ).wait()

                pltpu.sync_copy(
                    table_hbm.at[idx_v.at[pl.ds(step * _W, _W)]], buf)

                @pl.loop(0, _W, step=_L)
                def _(j):
                    rvec = j + lax.iota(jnp.int32, _L)
                    ivec = idx_v[pl.ds(step * _W + j, _L)]
                    tvec = tgt_v[pl.ds(step * _W + j, _L)]
                    lse_vals = plsc.load_gather(lse_v, [ivec])
                    elems = plsc.load_gather(buf, [rvec, tvec])
                    acc_v[pl.ds(0, _L)] = (
                        acc_v[pl.ds(0, _L)] + lse_vals - elems)

                pltpu.async_copy(
                    buf, out_hbm.at[pl.ds(base + step * _W, _W)], sem)

        for b in range(_RING):
            pltpu.make_async_copy(
                bufs[b],
                out_hbm.at[pl.ds(base + (steps - _RING + b) * _W, _W)],
                sems[b],
            ).wait()

        pltpu.sync_copy(acc_v, part_hbm.at[wid])

    return sc_main


def kernel(idx, targets, table):
    b, t = idx.shape
    v = table.shape[0]
    n = b * t
    idx_f = idx.reshape(n).astype(jnp.int32)
    tgt_f = targets.reshape(n).astype(jnp.int32)

    lse = pl.pallas_call(
        _lse_body,
        out_shape=jax.ShapeDtypeStruct((v, 1), jnp.float32),
    )(table)
    lse_p = jnp.pad(lse.reshape(v), (0, 1024 - v))
    table_p = jnp.pad(table, ((0, 0), (0, 1024 - v)))

    logits_p, partials = _make_sc_main(n, v)(table_p, idx_f, tgt_f, lse_p)
    logits = logits_p[:, :v]

    loss = pl.pallas_call(
        functools.partial(_loss_body, n),
        out_shape=jax.ShapeDtypeStruct((1, 1), jnp.float32),
    )(partials)

    return loss[0, 0], logits
